# baseline (speedup 1.0000x reference)
import jax
import jax.numpy as jnp
from jax import lax
from jax.experimental import pallas as pl
from jax.experimental.pallas import tpu as pltpu

B = 512
G = 1000
D = 128
BT = 32


def _fused_kernel(x_ref, ones_ref, emb_ref, feat_ref, ge_ref):
    x_blk = x_ref[...]
    e = emb_ref[...]
    ones2 = ones_ref[...]
    x_bf = x_blk.astype(jnp.bfloat16)
    for b in range(BT):
        row = x_bf[b : b + 1, :]
        bc = lax.dot_general(
            row, ones2, (((0,), (0,)), ((), ())),
            preferred_element_type=jnp.float32,
        )
        ge_ref[b] = bc * e
    feat_ref[...] = jnp.dot(x_blk, e, preferred_element_type=jnp.float32)


def kernel(x_dict, emb):
    ones2 = jnp.ones((1, D), jnp.bfloat16)
    grid = (B // BT,)
    feat, gene_emb = pl.pallas_call(
        _fused_kernel,
        grid=grid,
        in_specs=[
            pl.BlockSpec((BT, G), lambda i: (i, 0)),
            pl.BlockSpec((1, D), lambda i: (0, 0)),
            pl.BlockSpec((G, D), lambda i: (0, 0)),
        ],
        out_specs=[
            pl.BlockSpec((BT, D), lambda i: (i, 0)),
            pl.BlockSpec((BT, G, D), lambda i: (i, 0, 0)),
        ],
        out_shape=[
            jax.ShapeDtypeStruct((B, D), jnp.float32),
            jax.ShapeDtypeStruct((B, G, D), jnp.float32),
        ],
        compiler_params=pltpu.CompilerParams(
            dimension_semantics=("arbitrary",),
        ),
    )(x_dict, ones2, emb)
    return (feat, gene_emb)
